# Initial kernel scaffold; baseline (speedup 1.0000x reference)
import functools

import jax
import jax.numpy as jnp
from jax import lax
from jax.experimental import pallas as pl
from jax.experimental.pallas import tpu as pltpu
from jax.experimental.pallas import tpu_sc as plsc

NC = 2
NS = 16
NW = NC * NS
LANES = 16
CHUNK = 128

_MESH = plsc.VectorSubcoreMesh(core_axis_name="c", subcore_axis_name="s")


def _fill_const(ref, rows, value):
    cols = ref.shape[1]

    @pl.loop(0, rows)
    def _(i):
        @pl.loop(0, cols, step=LANES)
        def _(j):
            ref[i, pl.ds(j, LANES)] = jnp.full((LANES,), value, jnp.float32)


def _make_degree_kernel(n_idx, n_table):
    n_chunks = n_idx // CHUNK
    iters = (n_chunks + NW - 1) // NW
    rows_per_sub = n_table // NS
    zrows = 128
    n_zcopies = rows_per_sub // zrows
    z_tail = rows_per_sub - n_zcopies * zrows

    @functools.partial(
        pl.kernel,
        out_type=jax.ShapeDtypeStruct((NC * n_table, LANES), jnp.float32),
        mesh=_MESH,
        scratch_types=[
            pltpu.VMEM((CHUNK,), jnp.int32),
            pltpu.VMEM((CHUNK, LANES), jnp.float32),
            pltpu.VMEM((zrows, LANES), jnp.float32),
            pltpu.VMEM_SHARED((n_table, LANES), jnp.float32),
        ],
    )
    def deg_kernel(idx_hbm, out_hbm, idx_v, ones_v, zeros_v, table):
        cid = lax.axis_index("c")
        sid = lax.axis_index("s")
        wid = sid * NC + cid
        _fill_const(ones_v, CHUNK, 1.0)
        _fill_const(zeros_v, zrows, 0.0)
        base = sid * rows_per_sub

        @pl.loop(0, n_zcopies)
        def _(i):
            pltpu.sync_copy(zeros_v, table.at[pl.ds(base + i * zrows, zrows)])
        if z_tail:
            pltpu.sync_copy(zeros_v.at[pl.ds(0, z_tail)],
                            table.at[pl.ds(base + n_zcopies * zrows, z_tail)])
        plsc.subcore_barrier()

        @pl.loop(0, iters)
        def _(i):
            chunk = i * NW + wid

            @pl.when(chunk < n_chunks)
            def _():
                pltpu.sync_copy(idx_hbm.at[pl.ds(chunk * CHUNK, CHUNK)], idx_v)
                pltpu.sync_copy(ones_v, table.at[idx_v], add=True)

        plsc.subcore_barrier()
        pltpu.sync_copy(table.at[pl.ds(base, rows_per_sub)],
                        out_hbm.at[pl.ds(cid * n_table + base, rows_per_sub)])

    return deg_kernel


def _make_aggregate_kernel(n_nodes, n_edges, feat):
    n_chunks = n_edges // CHUNK
    iters = (n_chunks + NW - 1) // NW
    rows_per_sub = n_nodes // NS
    zrows = 128
    n_zcopies = rows_per_sub // zrows
    z_tail = rows_per_sub - n_zcopies * zrows

    @functools.partial(
        pl.kernel,
        out_type=jax.ShapeDtypeStruct((NC * n_nodes, feat), jnp.float32),
        mesh=_MESH,
        scratch_types=[
            pltpu.VMEM((CHUNK,), jnp.int32),
            pltpu.VMEM((CHUNK,), jnp.int32),
            pltpu.VMEM((CHUNK, feat), jnp.float32),
            pltpu.VMEM((zrows, feat), jnp.float32),
            pltpu.VMEM_SHARED((n_nodes, feat), jnp.float32),
            pltpu.SemaphoreType.DMA,
        ],
    )
    def agg_kernel(x_hbm, src_hbm, dst_hbm, out_hbm,
                   sidx_v, didx_v, rows_v, zeros_v, accum, sem):
        cid = lax.axis_index("c")
        sid = lax.axis_index("s")
        wid = sid * NC + cid
        _fill_const(zeros_v, zrows, 0.0)
        base = sid * rows_per_sub

        @pl.loop(0, n_zcopies)
        def _(i):
            pltpu.sync_copy(zeros_v, accum.at[pl.ds(base + i * zrows, zrows)])
        if z_tail:
            pltpu.sync_copy(zeros_v.at[pl.ds(0, z_tail)],
                            accum.at[pl.ds(base + n_zcopies * zrows, z_tail)])
        plsc.subcore_barrier()

        @pl.loop(0, iters)
        def _(i):
            chunk = i * NW + wid

            @pl.when(chunk < n_chunks)
            def _():
                pltpu.sync_copy(src_hbm.at[pl.ds(chunk * CHUNK, CHUNK)], sidx_v)
                pltpu.sync_copy(dst_hbm.at[pl.ds(chunk * CHUNK, CHUNK)], didx_v)
                pltpu.async_copy(x_hbm.at[sidx_v], rows_v, sem).wait()
                pltpu.sync_copy(rows_v, accum.at[didx_v], add=True)

        plsc.subcore_barrier()
        pltpu.sync_copy(accum.at[pl.ds(base, rows_per_sub)],
                        out_hbm.at[pl.ds(cid * n_nodes + base, rows_per_sub)])

    return agg_kernel


def _norm_from_degs(deg0, deg1):
    d = deg0[:, 0] + deg1[:, 0]
    return lax.rsqrt(jnp.maximum(d, 1.0))


def _tc_pre_body(x_ref, degs0_ref, degs1_ref, w_ref, o_ref):
    norm = _norm_from_degs(degs0_ref, degs1_ref)
    o_ref[...] = jnp.dot(x_ref[...] * norm[:, None], w_ref[...],
                         preferred_element_type=jnp.float32)


def _tc_mid_body(p0_ref, p1_ref, degd0_ref, degd1_ref, b_ref,
                 degs0_ref, degs1_ref, w_ref, o_ref):
    agg = p0_ref[...] + p1_ref[...]
    nd = _norm_from_degs(degd0_ref, degd1_ref)
    h = jnp.maximum(agg * nd[:, None] + b_ref[...], 0.0)
    ns = _norm_from_degs(degs0_ref, degs1_ref)
    o_ref[...] = jnp.dot(h * ns[:, None], w_ref[...],
                         preferred_element_type=jnp.float32)


def _tc_fin_body(p0_ref, p1_ref, degd0_ref, degd1_ref, b_ref, h_ref, c_ref):
    agg = p0_ref[...] + p1_ref[...]
    nd = _norm_from_degs(degd0_ref, degd1_ref)
    z = agg * nd[:, None] + b_ref[...]
    h = jax.nn.sigmoid(z)
    h_ref[...] = h
    c_ref[...] = jnp.where(h >= 0.5, 1.0, 0.0)


def kernel(in_feat, edge_index, W1, b1, W2, b2, W3, b3):
    n, f = in_feat.shape
    e = edge_index.shape[1]
    assert e % CHUNK == 0 and n % NS == 0 and f % 128 == 0

    src = edge_index[0]
    dst = edge_index[1]
    idx_all = jnp.concatenate([src, dst + n])

    degp = _make_degree_kernel(2 * e, 2 * n)(idx_all)
    degp = degp.reshape(NC, 2 * n, LANES)
    degp0, degp1 = degp[0], degp[1]

    agg_kernel = _make_aggregate_kernel(n, e, f)

    blk = 1000
    grid = (n // blk,)

    def deg_spec(part):
        off = (part * n) // blk
        return pl.BlockSpec((blk, LANES), lambda i, off=off: (i + off, 0))

    w_spec = pl.BlockSpec((f, f), lambda i: (0, 0))
    b_spec = pl.BlockSpec((1, f), lambda i: (0, 0))
    row_spec = pl.BlockSpec((blk, f), lambda i: (i, 0))

    tc_pre = pl.pallas_call(
        _tc_pre_body,
        out_shape=jax.ShapeDtypeStruct((n, f), jnp.float32),
        grid=grid,
        in_specs=[row_spec, deg_spec(0), deg_spec(0), w_spec],
        out_specs=row_spec,
    )

    tc_mid = pl.pallas_call(
        _tc_mid_body,
        out_shape=jax.ShapeDtypeStruct((n, f), jnp.float32),
        grid=grid,
        in_specs=[row_spec, row_spec, deg_spec(1), deg_spec(1), b_spec,
                  deg_spec(0), deg_spec(0), w_spec],
        out_specs=row_spec,
    )

    tc_fin = pl.pallas_call(
        _tc_fin_body,
        out_shape=(jax.ShapeDtypeStruct((n, f), jnp.float32),
                   jax.ShapeDtypeStruct((n, f), jnp.float32)),
        grid=grid,
        in_specs=[row_spec, row_spec, deg_spec(1), deg_spec(1), b_spec],
        out_specs=(row_spec, row_spec),
    )

    b1r = b1.reshape(1, f)
    b2r = b2.reshape(1, f)
    b3r = b3.reshape(1, f)

    feat1 = tc_pre(in_feat, degp0, degp1, W1)
    p = agg_kernel(feat1, src, dst).reshape(NC, n, f)
    feat2 = tc_mid(p[0], p[1], degp0, degp1, b1r, degp0, degp1, W2)
    p = agg_kernel(feat2, src, dst).reshape(NC, n, f)
    feat3 = tc_mid(p[0], p[1], degp0, degp1, b2r, degp0, degp1, W3)
    p = agg_kernel(feat3, src, dst).reshape(NC, n, f)
    h, h_clone = tc_fin(p[0], p[1], degp0, degp1, b3r)
    return (h, h_clone)


# speedup vs baseline: 5.4606x; 5.4606x over previous
import functools

import jax
import jax.numpy as jnp
from jax import lax
from jax.experimental import pallas as pl
from jax.experimental.pallas import tpu as pltpu
from jax.experimental.pallas import tpu_sc as plsc

NC = 2
NS = 16
NW = NC * NS
LANES = 16
CHUNK = 128

_MESH = plsc.VectorSubcoreMesh(core_axis_name="c", subcore_axis_name="s")


def _fill_const(ref, rows, value):
    cols = ref.shape[1]

    @pl.loop(0, rows)
    def _(i):
        @pl.loop(0, cols, step=LANES)
        def _(j):
            ref[i, pl.ds(j, LANES)] = jnp.full((LANES,), value, jnp.float32)


ZROWS = 80


def _strided_row_chunks(total_rows, worker, n_workers, body):
    n_chunks = total_rows // ZROWS
    iters = (n_chunks + n_workers - 1) // n_workers

    @pl.loop(0, iters)
    def _(i):
        c = i * n_workers + worker

        @pl.when(c < n_chunks)
        def _():
            body(c * ZROWS)


def _make_degree_kernel(n_edges, n_nodes, feat):
    n_chunks = n_edges // CHUNK
    iters = (n_chunks + NS - 1) // NS
    assert n_nodes % ZROWS == 0

    @functools.partial(
        pl.kernel,
        out_type=jax.ShapeDtypeStruct((NC * n_nodes, feat), jnp.float32),
        mesh=_MESH,
        scratch_types=[
            pltpu.VMEM((CHUNK,), jnp.int32),
            pltpu.VMEM((CHUNK, feat), jnp.float32),
            pltpu.VMEM((ZROWS, feat), jnp.float32),
            pltpu.VMEM_SHARED((n_nodes, feat), jnp.float32),
        ],
    )
    def deg_kernel(idx_hbm, out_hbm, idx_v, ones_v, zeros_v, table):
        cid = lax.axis_index("c")
        sid = lax.axis_index("s")
        _fill_const(ones_v, CHUNK, 1.0)
        _fill_const(zeros_v, ZROWS, 0.0)

        _strided_row_chunks(
            n_nodes, sid, NS,
            lambda r: pltpu.sync_copy(zeros_v, table.at[pl.ds(r, ZROWS)]))
        plsc.subcore_barrier()

        @pl.loop(0, iters)
        def _(i):
            chunk = i * NS + sid

            @pl.when(chunk < n_chunks)
            def _():
                pltpu.sync_copy(
                    idx_hbm.at[pl.ds(cid * n_edges + chunk * CHUNK, CHUNK)],
                    idx_v)
                pltpu.sync_copy(ones_v, table.at[idx_v], add=True)

        plsc.subcore_barrier()
        _strided_row_chunks(
            n_nodes, sid, NS,
            lambda r: pltpu.sync_copy(
                table.at[pl.ds(r, ZROWS)],
                out_hbm.at[pl.ds(cid * n_nodes + r, ZROWS)]))

    return deg_kernel


def _make_aggregate_kernel(n_nodes, n_edges, feat):
    n_chunks = n_edges // CHUNK
    iters = (n_chunks + NW - 1) // NW
    assert n_nodes % ZROWS == 0

    @functools.partial(
        pl.kernel,
        out_type=jax.ShapeDtypeStruct((NC * n_nodes, feat), jnp.float32),
        mesh=_MESH,
        scratch_types=[
            pltpu.VMEM((CHUNK,), jnp.int32),
            pltpu.VMEM((CHUNK,), jnp.int32),
            pltpu.VMEM((CHUNK, feat), jnp.float32),
            pltpu.VMEM((ZROWS, feat), jnp.float32),
            pltpu.VMEM_SHARED((n_nodes, feat), jnp.float32),
            pltpu.SemaphoreType.DMA,
        ],
    )
    def agg_kernel(x_hbm, src_hbm, dst_hbm, out_hbm,
                   sidx_v, didx_v, rows_v, zeros_v, accum, sem):
        cid = lax.axis_index("c")
        sid = lax.axis_index("s")
        wid = sid * NC + cid
        _fill_const(zeros_v, ZROWS, 0.0)

        _strided_row_chunks(
            n_nodes, sid, NS,
            lambda r: pltpu.sync_copy(zeros_v, accum.at[pl.ds(r, ZROWS)]))
        plsc.subcore_barrier()

        @pl.loop(0, iters)
        def _(i):
            chunk = i * NW + wid

            @pl.when(chunk < n_chunks)
            def _():
                pltpu.sync_copy(src_hbm.at[pl.ds(chunk * CHUNK, CHUNK)], sidx_v)
                pltpu.sync_copy(dst_hbm.at[pl.ds(chunk * CHUNK, CHUNK)], didx_v)
                pltpu.async_copy(x_hbm.at[sidx_v], rows_v, sem).wait()
                pltpu.sync_copy(rows_v, accum.at[didx_v], add=True)

        plsc.subcore_barrier()
        _strided_row_chunks(
            n_nodes, sid, NS,
            lambda r: pltpu.sync_copy(
                accum.at[pl.ds(r, ZROWS)],
                out_hbm.at[pl.ds(cid * n_nodes + r, ZROWS)]))

    return agg_kernel


def _norm_from_deg(deg_ref):
    return lax.rsqrt(jnp.maximum(deg_ref[:, 0], 1.0))


def _tc_pre_body(x_ref, degs_ref, w_ref, o_ref):
    norm = _norm_from_deg(degs_ref)
    o_ref[...] = jnp.dot(x_ref[...] * norm[:, None], w_ref[...],
                         preferred_element_type=jnp.float32)


def _tc_mid_body(p0_ref, p1_ref, degd_ref, b_ref, degs_ref, w_ref, o_ref):
    agg = p0_ref[...] + p1_ref[...]
    nd = _norm_from_deg(degd_ref)
    h = jnp.maximum(agg * nd[:, None] + b_ref[...], 0.0)
    ns = _norm_from_deg(degs_ref)
    o_ref[...] = jnp.dot(h * ns[:, None], w_ref[...],
                         preferred_element_type=jnp.float32)


def _tc_fin_body(p0_ref, p1_ref, degd_ref, b_ref, h_ref, c_ref):
    agg = p0_ref[...] + p1_ref[...]
    nd = _norm_from_deg(degd_ref)
    z = agg * nd[:, None] + b_ref[...]
    h = jax.nn.sigmoid(z)
    h_ref[...] = h
    c_ref[...] = jnp.where(h >= 0.5, 1.0, 0.0)


def kernel(in_feat, edge_index, W1, b1, W2, b2, W3, b3):
    n, f = in_feat.shape
    e = edge_index.shape[1]
    assert e % CHUNK == 0 and n % ZROWS == 0 and f % 128 == 0

    src = edge_index[0]
    dst = edge_index[1]
    idx_flat = edge_index.reshape(2 * e)

    degq = _make_degree_kernel(e, n, f)(idx_flat).reshape(NC, n, f)
    deg_src, deg_dst = degq[0], degq[1]

    agg_kernel = _make_aggregate_kernel(n, e, f)

    blk = 1000
    grid = (n // blk,)

    w_spec = pl.BlockSpec((f, f), lambda i: (0, 0))
    b_spec = pl.BlockSpec((1, f), lambda i: (0, 0))
    row_spec = pl.BlockSpec((blk, f), lambda i: (i, 0))

    tc_pre = pl.pallas_call(
        _tc_pre_body,
        out_shape=jax.ShapeDtypeStruct((n, f), jnp.float32),
        grid=grid,
        in_specs=[row_spec, row_spec, w_spec],
        out_specs=row_spec,
    )

    tc_mid = pl.pallas_call(
        _tc_mid_body,
        out_shape=jax.ShapeDtypeStruct((n, f), jnp.float32),
        grid=grid,
        in_specs=[row_spec, row_spec, row_spec, b_spec, row_spec, w_spec],
        out_specs=row_spec,
    )

    tc_fin = pl.pallas_call(
        _tc_fin_body,
        out_shape=(jax.ShapeDtypeStruct((n, f), jnp.float32),
                   jax.ShapeDtypeStruct((n, f), jnp.float32)),
        grid=grid,
        in_specs=[row_spec, row_spec, row_spec, b_spec],
        out_specs=(row_spec, row_spec),
    )

    b1r = b1.reshape(1, f)
    b2r = b2.reshape(1, f)
    b3r = b3.reshape(1, f)

    feat1 = tc_pre(in_feat, deg_src, W1)
    p = agg_kernel(feat1, src, dst).reshape(NC, n, f)
    feat2 = tc_mid(p[0], p[1], deg_dst, b1r, deg_src, W2)
    p = agg_kernel(feat2, src, dst).reshape(NC, n, f)
    feat3 = tc_mid(p[0], p[1], deg_dst, b2r, deg_src, W3)
    p = agg_kernel(feat3, src, dst).reshape(NC, n, f)
    h, h_clone = tc_fin(p[0], p[1], deg_dst, b3r)
    return (h, h_clone)
